# baseline (speedup 1.0000x reference)
import jax
import jax.numpy as jnp
from jax import lax
from jax.experimental import pallas as pl
from jax.experimental.pallas import tpu as pltpu

_N = 100000
_D = 128
_B = 4000
_NC = _N // _B
_NBUF = 8


def _body(keys_hbm, q_ref, out_ref, buf, sems):
    def _copy(c, par):
        return pltpu.make_async_copy(
            keys_hbm.at[pl.ds(c * _B, _B), :], buf.at[par], sems.at[par])

    for pre in range(_NBUF - 1):
        _copy(pre, pre).start()

    def _dist(block):
        return jnp.sum(jnp.abs(block - q_ref[...]), axis=1, keepdims=True)

    def chunk(c, carry):
        best8, bestc8 = carry
        par = lax.rem(c, _NBUF)

        @pl.when(c + _NBUF - 1 < _NC)
        def _prefetch():
            _copy(c + _NBUF - 1, lax.rem(c + _NBUF - 1, _NBUF)).start()

        _copy(c, par).wait()

        s = _dist(buf[par])
        m8 = jnp.min(s.reshape(_B // 8, 8, 1), axis=0)
        upd = m8 < best8
        best8 = jnp.where(upd, m8, best8)
        bestc8 = jnp.where(upd, jnp.full((8, 1), c, jnp.int32), bestc8)
        return best8, bestc8

    best8, bestc8 = lax.fori_loop(
        0, _NC, chunk,
        (jnp.full((8, 1), jnp.inf, jnp.float32),
         jnp.zeros((8, 1), jnp.int32)))

    m = jnp.min(best8)
    cwin = jnp.min(jnp.where(best8 == m, bestc8, jnp.int32(_NC)))

    rescan = pltpu.make_async_copy(
        keys_hbm.at[pl.ds(cwin * _B, _B), :], buf.at[0], sems.at[0])
    rescan.start()
    rescan.wait()

    s2 = _dist(buf[0])
    rows = cwin * _B + lax.broadcasted_iota(jnp.int32, (_B, 1), 0)
    out_ref[0] = jnp.min(jnp.where(s2 == m, rows, jnp.int32(_N)))


def kernel(keys, query):
    out = pl.pallas_call(
        _body,
        grid=(1,),
        in_specs=[
            pl.BlockSpec(memory_space=pltpu.MemorySpace.HBM),
            pl.BlockSpec((1, _D), lambda i: (0, 0)),
        ],
        out_specs=pl.BlockSpec(memory_space=pltpu.SMEM),
        out_shape=jax.ShapeDtypeStruct((1,), jnp.int32),
        scratch_shapes=[
            pltpu.VMEM((_NBUF, _B, _D), jnp.float32),
            pltpu.SemaphoreType.DMA((_NBUF,)),
        ],
    )(keys, query)
    return out[0]


# speedup vs baseline: 1.0236x; 1.0236x over previous
import jax
import jax.numpy as jnp
from jax import lax
from jax.experimental import pallas as pl
from jax.experimental.pallas import tpu as pltpu

_N = 100000
_D = 128
_B = 4000
_NC = _N // _B
_NBUF = 4


def _body(keys_hbm, q_ref, out_ref, buf, bestv_ref, bestp_ref, sbest_ref,
          sems):
    def _copy(c, par):
        return pltpu.make_async_copy(
            keys_hbm.at[pl.ds(c * _B, _B), :], buf.at[par], sems.at[par])

    for pre in range(_NBUF - 1):
        _copy(pre, pre).start()

    def chunk(c, carry):
        par = lax.rem(c, _NBUF)

        @pl.when(c + _NBUF - 1 < _NC)
        def _prefetch():
            _copy(c + _NBUF - 1, lax.rem(c + _NBUF - 1, _NBUF)).start()

        _copy(c, par).wait()

        s = jnp.sum(jnp.abs(buf[par] - q_ref[...]), axis=1, keepdims=True)
        m = jnp.min(s)
        take = jnp.logical_or(c == 0, m < bestv_ref[0])

        @pl.when(take)
        def _snapshot():
            bestv_ref[0] = m
            bestp_ref[0] = c
            sbest_ref[...] = s

        return carry

    lax.fori_loop(0, _NC, chunk, 0)

    rows = (bestp_ref[0] * _B
            + lax.broadcasted_iota(jnp.int32, (_B, 1), 0))
    out_ref[0] = jnp.min(
        jnp.where(sbest_ref[...] == bestv_ref[0], rows, jnp.int32(_N)))


def kernel(keys, query):
    out = pl.pallas_call(
        _body,
        grid=(1,),
        in_specs=[
            pl.BlockSpec(memory_space=pltpu.MemorySpace.HBM),
            pl.BlockSpec((1, _D), lambda i: (0, 0)),
        ],
        out_specs=pl.BlockSpec(memory_space=pltpu.SMEM),
        out_shape=jax.ShapeDtypeStruct((1,), jnp.int32),
        scratch_shapes=[
            pltpu.VMEM((_NBUF, _B, _D), jnp.float32),
            pltpu.SMEM((1,), jnp.float32),
            pltpu.SMEM((1,), jnp.int32),
            pltpu.VMEM((_B, 1), jnp.float32),
            pltpu.SemaphoreType.DMA((_NBUF,)),
        ],
    )(keys, query)
    return out[0]
